# Initial kernel scaffold; baseline (speedup 1.0000x reference)
import jax
import jax.numpy as jnp
from jax.experimental import pallas as pl


def kernel(nfeats, efeats, edge_index, W_ni, W_nj, W_fij, W_node, b_node, attn, bias_attn):
    raise NotImplementedError("write your pallas kernel here")


# speedup vs baseline: 5.1576x; 5.1576x over previous
import functools

import jax
import jax.numpy as jnp
from jax import lax
from jax.experimental import pallas as pl
from jax.experimental.pallas import tpu as pltpu
from jax.experimental.pallas import tpu_sc as plsc

F32 = jnp.float32
I32 = jnp.int32

NC = 2
NS = 16
NW = NC * NS
LANES = 16


def _bcast_lane(vec16, lane):
    idx = jnp.full((LANES, 1), lane, I32)
    dn = lax.GatherDimensionNumbers(offset_dims=(), collapsed_slice_dims=(0,),
                                    start_index_map=(0,))
    return lax.gather(vec16, idx, dn, (1,),
                      mode=lax.GatherScatterMode.PROMISE_IN_BOUNDS)


def _node_proj_tc(n_pad, f_in, f_out, h, fe):
    blk = 64
    grid = n_pad // blk

    def body(x_ref, wni_ref, wnj_ref, wnode_ref, b_ref, attn_ref,
             fni_ref, fnj_ref, hout_ref, sni_ref, snj_ref):
        x = x_ref[...]
        attn2 = attn_ref[...]
        pad = jnp.zeros((blk, 128 - h), F32)
        yni = jnp.dot(x, wni_ref[...], preferred_element_type=F32)
        ynj = jnp.dot(x, wnj_ref[...], preferred_element_type=F32)
        yh = jnp.dot(x, wnode_ref[...], preferred_element_type=F32) + b_ref[...]
        fni_ref[...] = yni
        fnj_ref[...] = ynj
        hout_ref[...] = yh
        s1 = jnp.sum(yni.reshape(blk, h, fe) * attn2[None], axis=-1)
        s2 = jnp.sum(ynj.reshape(blk, h, fe) * attn2[None], axis=-1)
        sni_ref[...] = jnp.concatenate([s1, pad], axis=1)
        snj_ref[...] = jnp.concatenate([s2, pad], axis=1)

    return pl.pallas_call(
        body,
        grid=(grid,),
        in_specs=[
            pl.BlockSpec((blk, f_in), lambda i: (i, 0)),
            pl.BlockSpec((f_in, f_out), lambda i: (0, 0)),
            pl.BlockSpec((f_in, f_out), lambda i: (0, 0)),
            pl.BlockSpec((f_in, f_out), lambda i: (0, 0)),
            pl.BlockSpec((1, f_out), lambda i: (0, 0)),
            pl.BlockSpec((h, fe), lambda i: (0, 0)),
        ],
        out_specs=[
            pl.BlockSpec((blk, f_out), lambda i: (i, 0)),
            pl.BlockSpec((blk, f_out), lambda i: (i, 0)),
            pl.BlockSpec((blk, f_out), lambda i: (i, 0)),
            pl.BlockSpec((blk, 128), lambda i: (i, 0)),
            pl.BlockSpec((blk, 128), lambda i: (i, 0)),
        ],
        out_shape=[
            jax.ShapeDtypeStruct((n_pad, f_out), F32),
            jax.ShapeDtypeStruct((n_pad, f_out), F32),
            jax.ShapeDtypeStruct((n_pad, f_out), F32),
            jax.ShapeDtypeStruct((n_pad, 128), F32),
            jax.ShapeDtypeStruct((n_pad, 128), F32),
        ],
    )


def _edge_proj_tc(e_pad, f_edge_in, f_out, h, fe):
    blk = 512
    grid = e_pad // blk

    def body(x_ref, w_ref, brow_ref, attn_ref, ffb_ref, sf_ref):
        y = jnp.dot(x_ref[...], w_ref[...], preferred_element_type=F32)
        y = y + brow_ref[...]
        ffb_ref[...] = y
        s = jnp.sum(y.reshape(blk, h, fe) * attn_ref[...][None], axis=-1)
        pad = jnp.zeros((blk, LANES - h), F32)
        sf_ref[...] = jnp.concatenate([s, pad], axis=1)

    return pl.pallas_call(
        body,
        grid=(grid,),
        in_specs=[
            pl.BlockSpec((blk, f_edge_in), lambda i: (i, 0)),
            pl.BlockSpec((f_edge_in, f_out), lambda i: (0, 0)),
            pl.BlockSpec((1, f_out), lambda i: (0, 0)),
            pl.BlockSpec((h, fe), lambda i: (0, 0)),
        ],
        out_specs=[
            pl.BlockSpec((blk, f_out), lambda i: (i, 0)),
            pl.BlockSpec((blk, LANES), lambda i: (i, 0)),
        ],
        out_shape=[
            jax.ShapeDtypeStruct((e_pad, f_out), F32),
            jax.ShapeDtypeStruct((e_pad, LANES), F32),
        ],
    )


def _sc_mesh():
    return plsc.VectorSubcoreMesh(core_axis_name="c", subcore_axis_name="s",
                                  num_cores=NC, num_subcores=NS)


def _edge_fout_sc(n_pad, e_pad, d, nch, c):
    @functools.partial(
        pl.kernel,
        out_type=[jax.ShapeDtypeStruct((e_pad, d), F32)],
        mesh=_sc_mesh(),
        scratch_types=[
            pltpu.VMEM((c, d), F32),
            pltpu.VMEM((c, d), F32),
            pltpu.VMEM((c, d), F32),
            pltpu.VMEM((c,), I32),
            pltpu.VMEM((c,), I32),
            pltpu.SemaphoreType.DMA,
        ],
    )
    def kern(fni, fnj, ffb, src_i, dst_i, fout,
             gni_v, gnj_v, ffb_v, src_v, dst_v, sem):
        cid = lax.axis_index("c")
        sid = lax.axis_index("s")
        wid = sid * NC + cid

        def chunk(it, carry):
            base = wid * (nch * c) + it * c
            pltpu.sync_copy(src_i.at[pl.ds(base, c)], src_v)
            pltpu.sync_copy(dst_i.at[pl.ds(base, c)], dst_v)
            pltpu.sync_copy(ffb.at[pl.ds(base, c)], ffb_v)
            pltpu.async_copy(fni.at[src_v], gni_v, sem).wait()
            pltpu.async_copy(fnj.at[dst_v], gnj_v, sem).wait()

            def frow(i, cy):
                for j in range(d // LANES):
                    sl = pl.ds(j * LANES, LANES)
                    gni_v[i, sl] = gni_v[i, sl] + gnj_v[i, sl] + ffb_v[i, sl]
                return cy
            lax.fori_loop(0, c, frow, 0)
            pltpu.sync_copy(gni_v, fout.at[pl.ds(base, c)])
            return carry

        lax.fori_loop(0, nch, chunk, 0)

    return kern


def _edge_denom_sc(n_pad, e_pad, h, nch, c):
    rpt = n_pad // NS

    @functools.partial(
        pl.kernel,
        out_type=[
            jax.ShapeDtypeStruct((e_pad, LANES), F32),
            jax.ShapeDtypeStruct((NC * n_pad, 128), F32),
        ],
        mesh=_sc_mesh(),
        scratch_types=[
            pltpu.VMEM((c,), I32),
            pltpu.VMEM((c,), I32),
            pltpu.VMEM((c, 128), F32),
            pltpu.VMEM((c, 128), F32),
            pltpu.VMEM((c, LANES), F32),
            pltpu.VMEM((c, LANES), F32),
            pltpu.VMEM((c, 128), F32),
            pltpu.VMEM_SHARED((n_pad, 128), F32),
            pltpu.SemaphoreType.DMA,
        ],
    )
    def kern(sni, snj, sfij, src_i, dst_i,
             exp_e, denom_p,
             src_v, dst_v, sA_v, sB_v, sF_v, exp16_v, expw_v, denom_sh, sem):
        cid = lax.axis_index("c")
        sid = lax.axis_index("s")
        wid = sid * NC + cid
        zero16 = jnp.zeros((LANES,), F32)

        def zrow(i, cy):
            for j in range(128 // LANES):
                expw_v[i, pl.ds(j * LANES, LANES)] = zero16
            return cy
        lax.fori_loop(0, c, zrow, 0)
        nfull, tail = rpt // c, rpt % c
        for b in range(nfull):
            pltpu.sync_copy(expw_v, denom_sh.at[pl.ds(sid * rpt + b * c, c)])
        if tail:
            pltpu.sync_copy(expw_v.at[pl.ds(0, tail)],
                            denom_sh.at[pl.ds(sid * rpt + nfull * c, tail)])
        plsc.subcore_barrier()

        def chunk(it, carry):
            base = wid * (nch * c) + it * c
            pltpu.sync_copy(src_i.at[pl.ds(base, c)], src_v)
            pltpu.sync_copy(dst_i.at[pl.ds(base, c)], dst_v)
            pltpu.sync_copy(sfij.at[pl.ds(base, c)], sF_v)
            pltpu.async_copy(sni.at[src_v], sA_v, sem).wait()
            pltpu.async_copy(snj.at[dst_v], sB_v, sem).wait()

            def erow(i, cy):
                sl = pl.ds(0, LANES)
                ev = sA_v[i, sl] + sB_v[i, sl] + sF_v[i, sl]
                ev = jnp.where(ev >= 0.0, ev, 0.2 * ev)
                ev = jnp.exp(ev)
                exp16_v[i, sl] = ev
                expw_v[i, sl] = ev
                return cy
            lax.fori_loop(0, c, erow, 0)
            pltpu.sync_copy(exp16_v, exp_e.at[pl.ds(base, c)])
            pltpu.sync_copy(expw_v, denom_sh.at[dst_v], add=True)
            return carry

        lax.fori_loop(0, nch, chunk, 0)
        plsc.subcore_barrier()
        pltpu.sync_copy(denom_sh.at[pl.ds(sid * rpt, rpt)],
                        denom_p.at[pl.ds(cid * n_pad + sid * rpt, rpt)])

    return kern


def _attn_coef_sc(n_pad, e_pad, h, nch, c):
    rpt = n_pad // NS

    @functools.partial(
        pl.kernel,
        out_type=[jax.ShapeDtypeStruct((e_pad, LANES), F32)],
        mesh=_sc_mesh(),
        scratch_types=[
            pltpu.VMEM((c,), I32),
            pltpu.VMEM((c, LANES), F32),
            pltpu.VMEM((c, 128), F32),
            pltpu.VMEM((c, LANES), F32),
            pltpu.VMEM((c, 128), F32),
            pltpu.VMEM((c, 128), F32),
            pltpu.VMEM_SHARED((n_pad, 128), F32),
            pltpu.SemaphoreType.DMA,
        ],
    )
    def kern(exp_e, dst_i, denom_p, a_e,
             dst_v, exp_v, d_v, a_v, t0, t1, dsum_sh, sem):
        cid = lax.axis_index("c")
        sid = lax.axis_index("s")
        wid = sid * NC + cid

        nfull, tail = rpt // c, rpt % c

        def sblock(off, cc):
            pltpu.sync_copy(denom_p.at[pl.ds(sid * rpt + off, cc)],
                            t0.at[pl.ds(0, cc)])
            pltpu.sync_copy(denom_p.at[pl.ds(n_pad + sid * rpt + off, cc)],
                            t1.at[pl.ds(0, cc)])

            def srow(i, cy):
                sl = pl.ds(0, LANES)
                t0[i, sl] = t0[i, sl] + t1[i, sl]
                return cy
            lax.fori_loop(0, cc, srow, 0)
            pltpu.sync_copy(t0.at[pl.ds(0, cc)],
                            dsum_sh.at[pl.ds(sid * rpt + off, cc)])

        for b in range(nfull):
            sblock(b * c, c)
        if tail:
            sblock(nfull * c, tail)
        plsc.subcore_barrier()

        def chunk(it, carry):
            base = wid * (nch * c) + it * c
            pltpu.sync_copy(dst_i.at[pl.ds(base, c)], dst_v)
            pltpu.sync_copy(exp_e.at[pl.ds(base, c)], exp_v)
            pltpu.async_copy(dsum_sh.at[dst_v], d_v, sem).wait()

            def arow(i, cy):
                sl = pl.ds(0, LANES)
                a_v[i, sl] = exp_v[i, sl] / d_v[i, sl]
                return cy
            lax.fori_loop(0, c, arow, 0)
            pltpu.sync_copy(a_v, a_e.at[pl.ds(base, c)])
            return carry

        lax.fori_loop(0, nch, chunk, 0)

    return kern


def _agg_kernel_sc(n_pad, e_pad, h, nch, c, nph, dph):
    rpt = n_pad // NS
    hph = h // nph
    fepl = dph // hph

    @functools.partial(
        pl.kernel,
        out_type=[jax.ShapeDtypeStruct((NC * nph * n_pad, dph), F32)],
        mesh=_sc_mesh(),
        scratch_types=[
            pltpu.VMEM((c, dph), F32),
            pltpu.VMEM((c,), I32),
            pltpu.VMEM((c,), I32),
            pltpu.VMEM((c,), I32),
            pltpu.VMEM((c, LANES), F32),
            pltpu.VMEM_SHARED((n_pad, dph), F32),
            pltpu.SemaphoreType.DMA,
        ],
    )
    def kern(hp, src_i, dst_i, a_e,
             res_p,
             hrow_v, src_v, dst_v, idxp_v, a_v, acc_sh, sem):
        cid = lax.axis_index("c")
        sid = lax.axis_index("s")
        wid = sid * NC + cid
        zero16 = jnp.zeros((LANES,), F32)

        for ph in range(nph):
            def zrow(i, cy):
                for j in range(dph // LANES):
                    hrow_v[i, pl.ds(j * LANES, LANES)] = zero16
                return cy
            lax.fori_loop(0, c, zrow, 0)
            nfull = rpt // c
            for b in range(nfull):
                pltpu.sync_copy(hrow_v, acc_sh.at[pl.ds(sid * rpt + b * c, c)])
            tail = rpt - nfull * c
            if tail:
                pltpu.sync_copy(hrow_v.at[pl.ds(0, tail)],
                                acc_sh.at[pl.ds(sid * rpt + nfull * c, tail)])
            plsc.subcore_barrier()

            def chunk(it, carry):
                base = wid * (nch * c) + it * c
                pltpu.sync_copy(src_i.at[pl.ds(base, c)], src_v)
                pltpu.sync_copy(dst_i.at[pl.ds(base, c)], dst_v)
                pltpu.sync_copy(a_e.at[pl.ds(base, c)], a_v)
                for k in range(c // LANES):
                    sl = pl.ds(k * LANES, LANES)
                    idxp_v[sl] = src_v[sl] * nph + ph
                pltpu.async_copy(hp.at[idxp_v], hrow_v, sem).wait()

                def mrow(i, cy):
                    arow = a_v[i, pl.ds(0, LANES)]
                    for j in range(dph // LANES):
                        head = ph * hph + (j * LANES) // fepl
                        m = _bcast_lane(arow, head)
                        sl = pl.ds(j * LANES, LANES)
                        hrow_v[i, sl] = hrow_v[i, sl] * m
                    return cy
                lax.fori_loop(0, c, mrow, 0)
                pltpu.sync_copy(hrow_v, acc_sh.at[dst_v], add=True)
                return carry

            lax.fori_loop(0, nch, chunk, 0)
            plsc.subcore_barrier()
            pltpu.sync_copy(
                acc_sh.at[pl.ds(sid * rpt, rpt)],
                res_p.at[pl.ds((cid * nph + ph) * n_pad + sid * rpt, rpt)])
            plsc.subcore_barrier()

    return kern


def _sum_partials_tc(n, n_pad, nph, dph):
    blk = 400
    grid_i = n // blk

    def body(p0_ref, p1_ref, out_ref):
        out_ref[...] = p0_ref[0] + p1_ref[0]

    return pl.pallas_call(
        body,
        grid=(nph, grid_i),
        in_specs=[
            pl.BlockSpec((1, blk, dph), lambda ph, i: (ph, i, 0)),
            pl.BlockSpec((1, blk, dph), lambda ph, i: (nph + ph, i, 0)),
        ],
        out_specs=pl.BlockSpec((blk, dph), lambda ph, i: (i, ph)),
        out_shape=jax.ShapeDtypeStruct((n, nph * dph), F32),
    )


def kernel(nfeats, efeats, edge_index, W_ni, W_nj, W_fij, W_node, b_node,
           attn, bias_attn):
    n, f_in = nfeats.shape
    e, f_edge_in = efeats.shape
    h, fe = bias_attn.shape
    d = h * fe
    fn = W_node.shape[1] // h

    c = 48
    n_pad = pl.cdiv(n, 128) * 128
    epc = NW * c
    e_pad = pl.cdiv(e, epc) * epc
    nch = e_pad // epc
    nph = 4
    dph = d // nph

    src = edge_index[0].astype(I32)
    dst = edge_index[1].astype(I32)
    src = jnp.concatenate([src, jnp.zeros((e_pad - e,), I32)])
    dst = jnp.concatenate([dst, jnp.full((e_pad - e,), n, I32)])

    attn2 = attn.reshape(h, fe)
    brow_node = b_node.reshape(1, d)
    brow_attn = bias_attn.reshape(1, d)

    nfeats_p = jnp.pad(nfeats, ((0, n_pad - n), (0, 0)))
    efeats_p = jnp.pad(efeats, ((0, e_pad - e), (0, 0)))
    f_ni, f_nj, h_out, s_ni, s_nj = _node_proj_tc(n_pad, f_in, d, h, fe)(
        nfeats_p, W_ni, W_nj, W_node, brow_node, attn2)
    ffb, s_fij = _edge_proj_tc(e_pad, f_edge_in, d, h, fe)(
        efeats_p, W_fij, brow_attn, attn2)

    fout, = _edge_fout_sc(n_pad, e_pad, d, nch, c)(
        f_ni, f_nj, ffb, src, dst)

    exp_e, denom_p = _edge_denom_sc(n_pad, e_pad, h, nch, c)(
        s_ni, s_nj, s_fij, src, dst)

    a_e, = _attn_coef_sc(n_pad, e_pad, h, nch, c)(exp_e, dst, denom_p)

    hp = h_out.reshape(n_pad * nph, dph)
    res_p, = _agg_kernel_sc(n_pad, e_pad, h, nch, c, nph, dph)(
        hp, src, dst, a_e)

    res = _sum_partials_tc(n, n_pad, nph, dph)(
        res_p.reshape(NC * nph, n_pad, dph),
        res_p.reshape(NC * nph, n_pad, dph))

    return res.reshape(n, h, fn), fout[:e].reshape(e, h, fe)
